# baseline (speedup 1.0000x reference)
import functools

import jax
import jax.numpy as jnp
from jax import lax
from jax.experimental import pallas as pl
from jax.experimental.pallas import tpu as pltpu
from jax.experimental.pallas import tpu_sc as plsc

D = 2048
H = 1024
K = 64
TOPK = 8

BM = 512
NW = 32
LANES = 16


def _mlp_body(x_ref, w1c_ref, b1c_ref, w2_ref, b2_ref, w3_ref, b3_ref, out_ref):
    x = x_ref[...]
    h = jnp.dot(x, w1c_ref[...], preferred_element_type=jnp.float32) + b1c_ref[...]
    h1 = jnp.maximum(h[:, :H], 0.0)
    xs = h[:, H:]
    h2 = jnp.maximum(
        jnp.dot(h1, w2_ref[...], preferred_element_type=jnp.float32)
        + b2_ref[...] + xs, 0.0)
    out_ref[...] = (jnp.dot(h2, w3_ref[...], preferred_element_type=jnp.float32)
                    + b3_ref[...])


def _logits(x, W1, b1, W2, b2, W3, b3, Ws):
    B = x.shape[0]
    w1c = jnp.concatenate([W1, Ws], axis=1)
    b1c = jnp.concatenate([b1, jnp.zeros_like(b1)])[None, :]
    return pl.pallas_call(
        _mlp_body,
        grid=(B // BM,),
        in_specs=[
            pl.BlockSpec((BM, D), lambda i: (i, 0)),
            pl.BlockSpec((D, 2 * H), lambda i: (0, 0)),
            pl.BlockSpec((1, 2 * H), lambda i: (0, 0)),
            pl.BlockSpec((H, H), lambda i: (0, 0)),
            pl.BlockSpec((1, H), lambda i: (0, 0)),
            pl.BlockSpec((H, K), lambda i: (0, 0)),
            pl.BlockSpec((1, K), lambda i: (0, 0)),
        ],
        out_specs=pl.BlockSpec((BM, K), lambda i: (i, 0)),
        out_shape=jax.ShapeDtypeStruct((B, K), jnp.float32),
    )(x, w1c, b1c, W2, b2[None, :], W3, b3[None, :])


def _route_body(ch, log_hbm, zero_hbm, out_hbm, log_v, out_v, sem0, sem1):
    wid = lax.axis_index("s") * 2 + lax.axis_index("c")
    base = wid * ch * K
    n = ch * K
    cp = pltpu.async_copy(log_hbm.at[pl.ds(base, n)], log_v, sem0)
    zp = pltpu.async_copy(zero_hbm, out_v, sem1)
    cp.wait()
    zp.wait()

    lanes = lax.iota(jnp.int32, LANES)
    neg = jnp.full((LANES,), -3.4e38, jnp.float32)
    zi = jnp.zeros((LANES,), jnp.int32)

    def group(g, carry):
        rowoff = (g * LANES + lanes) * K

        def col(j, c):
            tvs = list(c[:TOPK])
            tis = list(c[TOPK:])
            vi = lax.broadcast(j, (LANES,))
            v = plsc.load_gather(log_v, [rowoff + vi])
            for i in range(TOPK):
                swap = v > tvs[i]
                nv = jnp.where(swap, tvs[i], v)
                ni = jnp.where(swap, tis[i], vi)
                tvs[i] = jnp.where(swap, v, tvs[i])
                tis[i] = jnp.where(swap, vi, tis[i])
                v, vi = nv, ni
            return tuple(tvs) + tuple(tis)

        c = lax.fori_loop(0, K, col, (neg,) * TOPK + (zi,) * TOPK)
        tvs, tis = c[:TOPK], c[TOPK:]
        m = tvs[0]
        es = [jnp.exp(tvs[i] - m) for i in range(TOPK)]
        s = es[0]
        for i in range(1, TOPK):
            s = s + es[i]
        r = 1.0 / s
        for i in range(TOPK):
            plsc.store_scatter(out_v, [rowoff + tis[i]], es[i] * r)
        return carry

    lax.fori_loop(0, ch // LANES, group, 0)
    pltpu.sync_copy(out_v, out_hbm.at[pl.ds(base, n)])


def _route(logits):
    B = logits.shape[0]
    ch = B // NW
    mesh = plsc.VectorSubcoreMesh(core_axis_name="c", subcore_axis_name="s",
                                  num_cores=2, num_subcores=16)
    zeros = jnp.zeros((ch * K,), jnp.float32)
    run = pl.kernel(
        functools.partial(_route_body, ch),
        out_type=jax.ShapeDtypeStruct((B * K,), jnp.float32),
        mesh=mesh,
        compiler_params=pltpu.CompilerParams(needs_layout_passes=False),
        scratch_types=[
            pltpu.VMEM((ch * K,), jnp.float32),
            pltpu.VMEM((ch * K,), jnp.float32),
            pltpu.SemaphoreType.DMA,
            pltpu.SemaphoreType.DMA,
        ],
    )
    return run(logits.reshape(B * K), zeros).reshape(B, K)


def kernel(x, W1, b1, W2, b2, W3, b3, Ws):
    return _route(_logits(x, W1, b1, W2, b2, W3, b3, Ws))


# speedup vs baseline: 1.1273x; 1.1273x over previous
import functools

import jax
import jax.numpy as jnp
from jax import lax
from jax.experimental import pallas as pl
from jax.experimental.pallas import tpu as pltpu
from jax.experimental.pallas import tpu_sc as plsc

D = 2048
H = 1024
K = 64
TOPK = 8

BM = 1024
NW = 32
LANES = 16


def _mlp_body(x_ref, w1c_ref, b1c_ref, w2_ref, b2_ref, w3_ref, b3_ref, out_ref):
    x = x_ref[...]
    h = jnp.dot(x, w1c_ref[...], preferred_element_type=jnp.float32) + b1c_ref[...]
    h1 = jnp.maximum(h[:, :H], 0.0)
    xs = h[:, H:]
    h2 = jnp.maximum(
        jnp.dot(h1, w2_ref[...], preferred_element_type=jnp.float32)
        + b2_ref[...] + xs, 0.0)
    out_ref[...] = (jnp.dot(h2, w3_ref[...], preferred_element_type=jnp.float32)
                    + b3_ref[...])


def _logits(x, W1, b1, W2, b2, W3, b3, Ws):
    B = x.shape[0]
    w1c = jnp.concatenate([W1, Ws], axis=1)
    b1c = jnp.concatenate([b1, jnp.zeros_like(b1)])[None, :]
    return pl.pallas_call(
        _mlp_body,
        grid=(B // BM,),
        in_specs=[
            pl.BlockSpec((BM, D), lambda i: (i, 0)),
            pl.BlockSpec((D, 2 * H), lambda i: (0, 0)),
            pl.BlockSpec((1, 2 * H), lambda i: (0, 0)),
            pl.BlockSpec((H, H), lambda i: (0, 0)),
            pl.BlockSpec((1, H), lambda i: (0, 0)),
            pl.BlockSpec((H, K), lambda i: (0, 0)),
            pl.BlockSpec((1, K), lambda i: (0, 0)),
        ],
        out_specs=pl.BlockSpec((BM, K), lambda i: (i, 0)),
        out_shape=jax.ShapeDtypeStruct((B, K), jnp.float32),
        compiler_params=pltpu.CompilerParams(vmem_limit_bytes=128 * 1024 * 1024),
    )(x, w1c, b1c, W2, b2[None, :], W3, b3[None, :])


def _route_body(ch, log_hbm, zero_hbm, out_hbm, log_v, out_v, sem0, sem1):
    wid = lax.axis_index("s") * 2 + lax.axis_index("c")
    base = wid * ch * K
    n = ch * K
    cp = pltpu.async_copy(log_hbm.at[pl.ds(base, n)], log_v, sem0)
    zp = pltpu.async_copy(zero_hbm, out_v, sem1)
    cp.wait()
    zp.wait()

    lanes = lax.iota(jnp.int32, LANES)
    cols = [lanes + c * LANES for c in range(K // LANES)]
    sel = lanes < TOPK

    def merge(ak, av, bk, bv):
        rk = lax.rev(bk, (0,))
        rv = lax.rev(bv, (0,))
        c = ak > rk
        return jnp.where(c, ak, rk), jnp.where(c, av, rv)

    @plsc.parallel_loop(0, ch, unroll=4)
    def row_loop(r):
        off = r * K
        srt = [plsc.sort_key_val(log_v[pl.ds(off + c * LANES, LANES)],
                                 cols[c], descending=True)
               for c in range(K // LANES)]
        uk, uv = merge(*srt[0], *srt[1])
        vk, vv = merge(*srt[2], *srt[3])
        uk, uv = plsc.sort_key_val(uk, uv, descending=True)
        vk, vv = plsc.sort_key_val(vk, vv, descending=True)
        wk, wv = merge(uk, uv, vk, vv)
        wk, wv = plsc.sort_key_val(wk, wv, descending=True)
        m = jnp.max(wk)
        e = jnp.where(sel, jnp.exp(wk - m), 0.0)
        s = lax.broadcast(jnp.sum(e), (LANES,))
        plsc.store_scatter(out_v, [off + wv], e / s, mask=sel)

    pltpu.sync_copy(out_v, out_hbm.at[pl.ds(base, n)])


def _route(logits):
    B = logits.shape[0]
    ch = B // NW
    mesh = plsc.VectorSubcoreMesh(core_axis_name="c", subcore_axis_name="s",
                                  num_cores=2, num_subcores=16)
    zeros = jnp.zeros((ch * K,), jnp.float32)
    run = pl.kernel(
        functools.partial(_route_body, ch),
        out_type=jax.ShapeDtypeStruct((B * K,), jnp.float32),
        mesh=mesh,
        compiler_params=pltpu.CompilerParams(needs_layout_passes=False),
        scratch_types=[
            pltpu.VMEM((ch * K,), jnp.float32),
            pltpu.VMEM((ch * K,), jnp.float32),
            pltpu.SemaphoreType.DMA,
            pltpu.SemaphoreType.DMA,
        ],
    )
    return run(logits.reshape(B * K), zeros).reshape(B, K)


def kernel(x, W1, b1, W2, b2, W3, b3, Ws):
    return _route(_logits(x, W1, b1, W2, b2, W3, b3, Ws))
